# baseline (speedup 1.0000x reference)
import functools

import jax
import jax.numpy as jnp
from jax import lax
from jax.experimental import pallas as pl
from jax.experimental.pallas import tpu as pltpu
from jax.experimental.pallas import tpu_sc as plsc

EMB_DIM = 64
NUM_EMB = 1024
N_TOKENS = 16 * 32 * 32
BN = 2048


def _vq_body(x_ref, emb_ref, esq_ref, idx_ref, oh_ref):
    x = x_ref[...]
    emb = emb_ref[...]
    emb_sqr = esq_ref[...]
    z_sqr = jnp.sum(x * x, axis=1, keepdims=True)
    m2 = jax.lax.dot_general(
        x + x, emb, (((1,), (1,)), ((), ())),
        preferred_element_type=jnp.float32)
    dist = (emb_sqr + z_sqr) - m2
    dmin = jnp.min(dist, axis=1, keepdims=True)
    iotaf = jax.lax.broadcasted_iota(jnp.int32, dist.shape, 1
                                     ).astype(jnp.float32)
    idxf = jnp.min(jnp.where(dist == dmin, iotaf, float(NUM_EMB)),
                   axis=1, keepdims=True)
    idx = idxf[:, 0].astype(jnp.int32)
    oh = (iotaf == idxf).astype(jnp.float32)
    idx_ref[...] = idx
    oh_ref[...] = oh


_SC_INFO = plsc.get_sparse_core_info()
_NW = _SC_INFO.num_cores * _SC_INFO.num_subcores
_B_PER_W = N_TOKENS // _NW


def _sc_gather_body(table_hbm, idx_hbm, out_hbm, idx_v, rows_v, sem):
    wid = lax.axis_index("s") * _SC_INFO.num_cores + lax.axis_index("c")
    base = wid * _B_PER_W
    pltpu.sync_copy(idx_hbm.at[pl.ds(base, _B_PER_W)], idx_v)
    pltpu.async_copy(table_hbm.at[idx_v], rows_v, sem).wait()
    pltpu.sync_copy(rows_v, out_hbm.at[pl.ds(base, _B_PER_W)])


@functools.partial(jax.jit, static_argnames=())
def kernel(z_e, embedding):
    z = jnp.transpose(z_e, (0, 2, 3, 1))
    z_flat = z.reshape(-1, EMB_DIM)
    emb_sqr = jnp.sum(embedding ** 2, axis=1).reshape(1, NUM_EMB)
    grid = (N_TOKENS // BN,)
    idx, oh = pl.pallas_call(
        _vq_body,
        grid=grid,
        in_specs=[
            pl.BlockSpec((BN, EMB_DIM), lambda i: (i, 0)),
            pl.BlockSpec((NUM_EMB, EMB_DIM), lambda i: (0, 0)),
            pl.BlockSpec((1, NUM_EMB), lambda i: (0, 0)),
        ],
        out_specs=[
            pl.BlockSpec((BN,), lambda i: (i,)),
            pl.BlockSpec((BN, NUM_EMB), lambda i: (i, 0)),
        ],
        out_shape=[
            jax.ShapeDtypeStruct((N_TOKENS,), jnp.int32),
            jax.ShapeDtypeStruct((N_TOKENS, NUM_EMB), jnp.float32),
        ],
    )(z_flat, embedding, emb_sqr)
    mesh = plsc.VectorSubcoreMesh(core_axis_name="c", subcore_axis_name="s")
    zq = pl.kernel(
        _sc_gather_body,
        mesh=mesh,
        compiler_params=pltpu.CompilerParams(use_tc_tiling_on_sc=False),
        out_type=jax.ShapeDtypeStruct((N_TOKENS, EMB_DIM), jnp.float32),
        scratch_types=[
            pltpu.VMEM((_B_PER_W,), jnp.int32),
            pltpu.VMEM((_B_PER_W, EMB_DIM), jnp.float32),
            pltpu.SemaphoreType.DMA,
        ],
    )(embedding, idx)
    z_q = zq.reshape(z.shape)
    return (z, z_q, idx, oh)


# speedup vs baseline: 1.2822x; 1.2822x over previous
import functools

import jax
import jax.numpy as jnp
from jax.experimental import pallas as pl

EMB_DIM = 64
NUM_EMB = 1024
N_TOKENS = 16 * 32 * 32
BN = 1024


def _vq_body(x_ref, emb_ref, esq_ref, idx_ref, oh_ref, zq_ref):
    x = x_ref[...]
    emb = emb_ref[...]
    emb_sqr = esq_ref[...]
    z_sqr = jnp.sum(x * x, axis=1, keepdims=True)
    m2 = jax.lax.dot_general(
        x + x, emb, (((1,), (1,)), ((), ())),
        preferred_element_type=jnp.float32)
    dist = (emb_sqr + z_sqr) - m2
    dmin = jnp.min(dist, axis=1, keepdims=True)
    iotaf = jax.lax.broadcasted_iota(jnp.int32, dist.shape, 1
                                     ).astype(jnp.float32)
    masked = jnp.where(dist == dmin, iotaf, float(NUM_EMB))
    idxf = jnp.min(masked, axis=1, keepdims=True)
    idx = idxf[:, 0].astype(jnp.int32)
    oh = (masked == idxf).astype(jnp.float32)
    zq = jax.lax.dot_general(
        oh, emb, (((1,), (0,)), ((), ())),
        preferred_element_type=jnp.float32)
    idx_ref[...] = idx
    oh_ref[...] = oh
    zq_ref[...] = zq


@functools.partial(jax.jit, static_argnames=())
def kernel(z_e, embedding):
    z = jnp.transpose(z_e, (0, 2, 3, 1))
    z_flat = z.reshape(-1, EMB_DIM)
    emb_sqr = jnp.sum(embedding ** 2, axis=1).reshape(1, NUM_EMB)
    grid = (N_TOKENS // BN,)
    idx, oh, zq = pl.pallas_call(
        _vq_body,
        grid=grid,
        in_specs=[
            pl.BlockSpec((BN, EMB_DIM), lambda i: (i, 0)),
            pl.BlockSpec((NUM_EMB, EMB_DIM), lambda i: (0, 0)),
            pl.BlockSpec((1, NUM_EMB), lambda i: (0, 0)),
        ],
        out_specs=[
            pl.BlockSpec((BN,), lambda i: (i,)),
            pl.BlockSpec((BN, NUM_EMB), lambda i: (i, 0)),
            pl.BlockSpec((BN, EMB_DIM), lambda i: (i, 0)),
        ],
        out_shape=[
            jax.ShapeDtypeStruct((N_TOKENS,), jnp.int32),
            jax.ShapeDtypeStruct((N_TOKENS, NUM_EMB), jnp.float32),
            jax.ShapeDtypeStruct((N_TOKENS, EMB_DIM), jnp.float32),
        ],
    )(z_flat, embedding, emb_sqr)
    z_q = zq.reshape(z.shape)
    return (z, z_q, idx, oh)


# speedup vs baseline: 1.3025x; 1.0159x over previous
import functools

import jax
import jax.numpy as jnp
from jax.experimental import pallas as pl
from jax.experimental.pallas import tpu as pltpu

EMB_DIM = 64
NUM_EMB = 1024
N_TOKENS = 16 * 32 * 32
BN = 4096


def _vq_body(x_ref, emb_ref, esq_ref, idx_ref, oh_ref, zq_ref):
    x = x_ref[...]
    emb = emb_ref[...]
    emb_sqr = esq_ref[...]
    z_sqr = jnp.sum(x * x, axis=1, keepdims=True)
    m2 = jax.lax.dot_general(
        x + x, emb, (((1,), (1,)), ((), ())),
        preferred_element_type=jnp.float32)
    dist = (emb_sqr + z_sqr) - m2
    dmin = jnp.min(dist, axis=1, keepdims=True)
    iotaf = jax.lax.broadcasted_iota(jnp.int32, dist.shape, 1
                                     ).astype(jnp.float32)
    masked = jnp.where(dist == dmin, iotaf, float(NUM_EMB))
    idxf = jnp.min(masked, axis=1, keepdims=True)
    idx = idxf[:, 0].astype(jnp.int32)
    oh = (masked == idxf).astype(jnp.float32)
    zq = jax.lax.dot_general(
        oh, emb, (((1,), (0,)), ((), ())),
        preferred_element_type=jnp.float32)
    idx_ref[...] = idx
    oh_ref[...] = oh
    zq_ref[...] = zq


@functools.partial(jax.jit, static_argnames=())
def kernel(z_e, embedding):
    z = jnp.transpose(z_e, (0, 2, 3, 1))
    z_flat = z.reshape(-1, EMB_DIM)
    emb_sqr = jnp.sum(embedding ** 2, axis=1).reshape(1, NUM_EMB)
    grid = (N_TOKENS // BN,)
    idx, oh, zq = pl.pallas_call(
        _vq_body,
        grid=grid,
        compiler_params=pltpu.CompilerParams(
            vmem_limit_bytes=63 * 1024 * 1024),
        in_specs=[
            pl.BlockSpec((BN, EMB_DIM), lambda i: (i, 0)),
            pl.BlockSpec((NUM_EMB, EMB_DIM), lambda i: (0, 0)),
            pl.BlockSpec((1, NUM_EMB), lambda i: (0, 0)),
        ],
        out_specs=[
            pl.BlockSpec((BN,), lambda i: (i,)),
            pl.BlockSpec((BN, NUM_EMB), lambda i: (i, 0)),
            pl.BlockSpec((BN, EMB_DIM), lambda i: (i, 0)),
        ],
        out_shape=[
            jax.ShapeDtypeStruct((N_TOKENS,), jnp.int32),
            jax.ShapeDtypeStruct((N_TOKENS, NUM_EMB), jnp.float32),
            jax.ShapeDtypeStruct((N_TOKENS, EMB_DIM), jnp.float32),
        ],
    )(z_flat, embedding, emb_sqr)
    z_q = zq.reshape(z.shape)
    return (z, z_q, idx, oh)


# speedup vs baseline: 1.3347x; 1.0248x over previous
import functools

import jax
import jax.numpy as jnp
from jax.experimental import pallas as pl

EMB_DIM = 64
NUM_EMB = 1024
N_TOKENS = 16 * 32 * 32
BN = 2048


def _vq_body(x_ref, emb_ref, esq_ref, idx_ref, oh_ref, zq_ref):
    x = x_ref[...]
    emb = emb_ref[...]
    emb_sqr = esq_ref[...]
    z_sqr = jnp.sum(x * x, axis=1, keepdims=True)
    m2 = jax.lax.dot_general(
        x + x, emb, (((1,), (1,)), ((), ())),
        preferred_element_type=jnp.float32)
    dist = (emb_sqr + z_sqr) - m2
    dmin = jnp.min(dist, axis=1, keepdims=True)
    iotaf = jax.lax.broadcasted_iota(jnp.int32, dist.shape, 1
                                     ).astype(jnp.float32)
    idxf = jnp.min(jnp.where(dist == dmin, iotaf, float(NUM_EMB)),
                   axis=1, keepdims=True)
    idx = idxf[:, 0].astype(jnp.int32)
    oh = (iotaf == idxf).astype(jnp.float32)
    zq = jax.lax.dot_general(
        oh, emb, (((1,), (0,)), ((), ())),
        preferred_element_type=jnp.float32)
    idx_ref[...] = idx
    oh_ref[...] = oh
    zq_ref[...] = zq


@functools.partial(jax.jit, static_argnames=())
def kernel(z_e, embedding):
    z = jnp.transpose(z_e, (0, 2, 3, 1))
    z_flat = z.reshape(-1, EMB_DIM)
    emb_sqr = jnp.sum(embedding ** 2, axis=1).reshape(1, NUM_EMB)
    grid = (N_TOKENS // BN,)
    idx, oh, zq = pl.pallas_call(
        _vq_body,
        grid=grid,
        in_specs=[
            pl.BlockSpec((BN, EMB_DIM), lambda i: (i, 0)),
            pl.BlockSpec((NUM_EMB, EMB_DIM), lambda i: (0, 0)),
            pl.BlockSpec((1, NUM_EMB), lambda i: (0, 0)),
        ],
        out_specs=[
            pl.BlockSpec((BN,), lambda i: (i,)),
            pl.BlockSpec((BN, NUM_EMB), lambda i: (i, 0)),
            pl.BlockSpec((BN, EMB_DIM), lambda i: (i, 0)),
        ],
        out_shape=[
            jax.ShapeDtypeStruct((N_TOKENS,), jnp.int32),
            jax.ShapeDtypeStruct((N_TOKENS, NUM_EMB), jnp.float32),
            jax.ShapeDtypeStruct((N_TOKENS, EMB_DIM), jnp.float32),
        ],
    )(z_flat, embedding, emb_sqr)
    z_q = zq.reshape(z.shape)
    return (z, z_q, idx, oh)
